# Initial kernel scaffold; baseline (speedup 1.0000x reference)
import jax
import jax.numpy as jnp
from jax.experimental import pallas as pl


def kernel(h, x, edge_index, emb_W, emb_b, eW1, eb1, eW2, eb2, nW1, nb1, nW2, nb2, wW1, wb1, wW2, wb2):
    raise NotImplementedError("write your pallas kernel here")


# speedup vs baseline: 2.3217x; 2.3217x over previous
import jax
import jax.numpy as jnp
import numpy as np
from jax import lax
from jax.experimental import pallas as pl
from jax.experimental.pallas import tpu as pltpu
from jax.experimental.pallas import tpu_sc as plsc

N = 10000
E = 320000
H = 128

NC = 2
NS = 16
NW = NC * NS
C = 256
IB = C // 128
EP = 327680
NCHUNK = EP // C // NW
NP = 10240
SROWS = NP // NS
EB = 512
NB = 1000

_f32 = jnp.float32
_i32 = jnp.int32
_bf16 = jnp.bfloat16


def _atan01(q):
    s = q * q
    return q * (0.99997726 + s * (-0.33262347 + s * (0.19354346 + s * (
        -0.11643287 + s * (0.05265332 + s * (-0.01172120))))))


def _k1_body(h_ref, embW_ref, embb_ref, w1a_ref, w1b_ref,
             he_ref, a_ref, b_ref):
    he = jnp.dot(h_ref[...].astype(_bf16), embW_ref[...].astype(_bf16),
                 preferred_element_type=_f32)
    he = he + embb_ref[...]
    he_ref[...] = he
    heb = he.astype(_bf16)
    a_ref[...] = jnp.dot(heb, w1a_ref[...].astype(_bf16),
                         preferred_element_type=_f32)
    b_ref[...] = jnp.dot(heb, w1b_ref[...].astype(_bf16),
                         preferred_element_type=_f32)


def _k1(h, emb_W, emb_b, w1a, w1b):
    blk = pl.BlockSpec((NB, H), lambda i: (i, 0))
    wblk = pl.BlockSpec((H, H), lambda i: (0, 0))
    vblk = pl.BlockSpec((1, H), lambda i: (0, 0))
    return pl.pallas_call(
        _k1_body,
        grid=(N // NB,),
        in_specs=[blk, wblk, vblk, wblk, wblk],
        out_specs=[blk, blk, blk],
        out_shape=[jax.ShapeDtypeStruct((N, H), _f32)] * 3,
    )(h, emb_W, emb_b, w1a, w1b)


def _k2_body(a_hbm, b_hbm, x0_hbm, x1_hbm, x2_hbm, row_hbm, col_hbm,
             g_hbm, d_hbm, c2_hbm, s2_hbm, ph_hbm,
             idxr, idxc, a_v, b_v, x0_v, x1_v, x2_v,
             d_v, c2_v, s2_v, ph_v, sem):
    wid = lax.axis_index("s") * NC + lax.axis_index("c")
    pltpu.sync_copy(x0_hbm, x0_v)
    pltpu.sync_copy(x1_hbm, x1_v)
    pltpu.sync_copy(x2_hbm, x2_v)
    lanes = lax.iota(_i32, 16)

    def chunk_body(t, carry):
        chunk = wid * NCHUNK + t
        e0 = chunk * C
        r0 = chunk * IB
        pltpu.sync_copy(row_hbm.at[pl.ds(r0, IB)], idxr)
        pltpu.sync_copy(col_hbm.at[pl.ds(r0, IB)], idxc)
        for j in range(IB):
            sl = pl.ds(j * 128, 128)
            pltpu.async_copy(a_hbm.at[idxr.at[j]], a_v.at[sl], sem).wait()
            pltpu.async_copy(b_hbm.at[idxc.at[j]], b_v.at[sl], sem).wait()

        for j in range(IB):
            def geo_body(gg, c2carry, _j=j):
                off = gg * 16
                ir = idxr[_j, pl.ds(off, 16)]
                ic = idxc[_j, pl.ds(off, 16)]
                xd = plsc.load_gather(x0_v, [ir]) - plsc.load_gather(x0_v, [ic])
                yd = plsc.load_gather(x1_v, [ir]) - plsc.load_gather(x1_v, [ic])
                zd = plsc.load_gather(x2_v, [ir]) - plsc.load_gather(x2_v, [ic])
                x2 = xd * xd
                y2 = yd * yd
                rxy2 = x2 + y2
                d = rxy2 + zd * zd
                safe = jnp.maximum(rxy2, 1e-30)
                pos = rxy2 > 0.0
                c2 = jnp.where(pos, (x2 - y2) / safe, 1.0)
                s2 = jnp.where(pos, (2.0 * xd * yd) / safe, 0.0)
                ai = plsc.bitcast(safe, _i32)
                y = plsc.bitcast(_i32(0x5F3759DF) - (ai >> 1), _f32)
                y = y * (1.5 - 0.5 * safe * y * y)
                y = y * (1.5 - 0.5 * safe * y * y)
                y = y * (1.5 - 0.5 * safe * y * y)
                rxy = safe * y
                az = jnp.abs(zd)
                mx = jnp.maximum(az, rxy)
                mn = jnp.minimum(az, rxy)
                q = mn / jnp.maximum(mx, 1e-30)
                a = _atan01(q)
                phi = jnp.sign(zd) * jnp.where(az > rxy, np.float32(np.pi / 2) - a, a)
                base = _j * 128 + off
                d_v[pl.ds(base, 16)] = d
                c2_v[pl.ds(base, 16)] = c2
                s2_v[pl.ds(base, 16)] = s2
                ph_v[pl.ds(base, 16)] = phi
                return c2carry

            lax.fori_loop(0, 8, geo_body, 0)

        def add_body(i, c2carry):
            for jj in range(8):
                s2l = pl.ds(jj * 16, 16)
                a_v[i, s2l] = a_v[i, s2l] + b_v[i, s2l]
            return c2carry

        lax.fori_loop(0, C, add_body, 0)
        pltpu.sync_copy(a_v, g_hbm.at[pl.ds(e0, C)])
        pltpu.sync_copy(d_v, d_hbm.at[pl.ds(e0, C)])
        pltpu.sync_copy(c2_v, c2_hbm.at[pl.ds(e0, C)])
        pltpu.sync_copy(s2_v, s2_hbm.at[pl.ds(e0, C)])
        pltpu.sync_copy(ph_v, ph_hbm.at[pl.ds(e0, C)])
        return carry

    lax.fori_loop(0, NCHUNK, chunk_body, 0)
    del lanes


def _k2(A, B, x0, x1, x2, row2, col2):
    mesh = plsc.VectorSubcoreMesh(core_axis_name="c", subcore_axis_name="s")
    return pl.kernel(
        _k2_body,
        out_type=[jax.ShapeDtypeStruct((EP, H), _f32),
                  jax.ShapeDtypeStruct((EP,), _f32),
                  jax.ShapeDtypeStruct((EP,), _f32),
                  jax.ShapeDtypeStruct((EP,), _f32),
                  jax.ShapeDtypeStruct((EP,), _f32)],
        mesh=mesh,
        compiler_params=pltpu.CompilerParams(needs_layout_passes=False),
        scratch_types=[
            pltpu.VMEM((IB, 128), _i32),
            pltpu.VMEM((IB, 128), _i32),
            pltpu.VMEM((C, H), _f32),
            pltpu.VMEM((C, H), _f32),
            pltpu.VMEM((N,), _f32),
            pltpu.VMEM((N,), _f32),
            pltpu.VMEM((N,), _f32),
            pltpu.VMEM((C,), _f32),
            pltpu.VMEM((C,), _f32),
            pltpu.VMEM((C,), _f32),
            pltpu.VMEM((C,), _f32),
            pltpu.SemaphoreType.DMA,
        ],
    )(A, B, x0, x1, x2, row2, col2)


def _k3_body(g_ref, d_ref, c2_ref, s2_ref, ph_ref, u_ref, b1_ref,
             w2_ref, b2_ref, ww1_ref, wb1_ref, wv2_ref, wb2_ref,
             f_ref, m0_ref, m1_ref, m2_ref):
    ident = (lax.broadcasted_iota(_i32, (128, 128), 0) ==
             lax.broadcasted_iota(_i32, (128, 128), 1)).astype(_f32)

    def unpack_col(ref):
        pk = jnp.reshape(ref[...], (EB // 128, 128))
        cols = [jnp.sum(jnp.broadcast_to(pk[i:i + 1, :], (128, 128)) * ident,
                        axis=1, keepdims=True) for i in range(EB // 128)]
        return jnp.concatenate(cols, axis=0)

    def pack_row(colv):
        rows = [jnp.sum(colv[i * 128:(i + 1) * 128] * ident,
                        axis=0, keepdims=True) for i in range(EB // 128)]
        return jnp.reshape(jnp.concatenate(rows, axis=0), (1, EB // 128, 128))

    d = unpack_col(d_ref)
    c2 = unpack_col(c2_ref)
    s2 = unpack_col(s2_ref)
    ph = unpack_col(ph_ref)

    db = d.astype(_bf16).astype(_f32)
    ub = u_ref[...].astype(_bf16).astype(_f32)
    pre = g_ref[...] + db * ub + b1_ref[...]
    t1 = jnp.maximum(pre, 0.0)
    F0 = jnp.dot(t1.astype(_bf16), w2_ref[...].astype(_bf16),
                 preferred_element_type=_f32) + b2_ref[...]
    t2 = jnp.maximum(
        jnp.dot(F0.astype(_bf16), ww1_ref[...].astype(_bf16),
                preferred_element_type=_f32) + wb1_ref[...],
        0.0)
    t2b = t2.astype(_bf16).astype(_f32)
    wv2b = wv2_ref[...].astype(_bf16).astype(_f32)
    w = jnp.sum(t2b * wv2b, axis=1, keepdims=True) + wb2_ref[:, 0:1]

    eids = pl.program_id(0) * EB + lax.broadcasted_iota(_i32, (EB, 1), 0)
    valid = eids < E
    f_ref[...] = jnp.where(valid, F0, 0.0)
    dw = jnp.where(valid, d * w, 0.0)
    m0_ref[...] = pack_row(dw * c2)
    m1_ref[...] = pack_row(dw * s2)
    m2_ref[...] = pack_row(dw * ph)


def _k3(G, D2, C22, S22, PH2, u, b1, eW2, b2, wW1, wb1, wv2, wb2):
    eblk = pl.BlockSpec((EB, H), lambda i: (i, 0))
    nblk = pl.BlockSpec((1, EB // 128, 128), lambda i: (i, 0, 0))
    wblk = pl.BlockSpec((H, H), lambda i: (0, 0))
    vblk = pl.BlockSpec((1, H), lambda i: (0, 0))
    nsh = jax.ShapeDtypeStruct((EP // EB, EB // 128, 128), _f32)
    return pl.pallas_call(
        _k3_body,
        grid=(EP // EB,),
        in_specs=[eblk, nblk, nblk, nblk, nblk,
                  vblk, vblk, wblk, vblk, wblk, vblk, vblk, vblk],
        out_specs=[eblk, nblk, nblk, nblk],
        out_shape=[jax.ShapeDtypeStruct((EP, H), _f32), nsh, nsh, nsh],
    )(G, D2, C22, S22, PH2, u, b1, eW2, b2, wW1, wb1, wv2, wb2)


C4 = 128
NCHUNK4 = EP // C4 // NW


def _k4a_body(f_hbm, row_hbm, p_hbm, idx, f_v, accF, sem):
    cid = lax.axis_index("c")
    sid = lax.axis_index("s")
    wid = sid * NC + cid
    s0 = sid * SROWS

    zero16 = jnp.zeros((16,), _f32)

    def zero_fv(i, carry):
        for jj in range(8):
            f_v[i, pl.ds(jj * 16, 16)] = zero16
        return carry

    lax.fori_loop(0, C4, zero_fv, 0)

    def zero_accf(k, carry):
        pltpu.sync_copy(f_v.at[pl.ds(0, SROWS // 8)],
                        accF.at[pl.ds(s0 + k * (SROWS // 8), SROWS // 8)])
        return carry

    lax.fori_loop(0, 8, zero_accf, 0)
    plsc.subcore_barrier()

    def chunk_body(t, carry):
        chunk = wid * NCHUNK4 + t
        e0 = chunk * C4
        pltpu.sync_copy(row_hbm.at[pl.ds(chunk, 1)], idx)
        pltpu.sync_copy(f_hbm.at[pl.ds(e0, C4)], f_v)
        pltpu.sync_copy(f_v, accF.at[idx.at[0]], add=True)
        return carry

    lax.fori_loop(0, NCHUNK4, chunk_body, 0)
    plsc.subcore_barrier()

    def fgroup(g, carry):
        r0 = s0 + g * C4
        pltpu.sync_copy(accF.at[pl.ds(r0, C4)], f_v)
        out0 = pl.multiple_of(cid * NP + r0, C4)
        pltpu.sync_copy(f_v, p_hbm.at[pl.ds(out0, C4)])
        return carry

    lax.fori_loop(0, SROWS // C4, fgroup, 0)


def _k4a(F, row2):
    mesh = plsc.VectorSubcoreMesh(core_axis_name="c", subcore_axis_name="s")
    return pl.kernel(
        _k4a_body,
        out_type=jax.ShapeDtypeStruct((2 * NP, H), _f32),
        mesh=mesh,
        compiler_params=pltpu.CompilerParams(needs_layout_passes=False),
        scratch_types=[
            pltpu.VMEM((1, 128), _i32),
            pltpu.VMEM((C4, H), _f32),
            pltpu.VMEM_SHARED((NP, H), _f32),
            pltpu.SemaphoreType.DMA,
        ],
    )(F, row2)


def _k4b_body(m0_hbm, m1_hbm, m2_hbm, row_hbm, pm_hbm,
              idx, m_v, m0_v, m1_v, m2_v, accM, sem):
    cid = lax.axis_index("c")
    sid = lax.axis_index("s")
    wid = sid * NC + cid
    s0 = sid * SROWS

    zero16 = jnp.zeros((16,), _f32)
    ones16 = jnp.ones((16,), _f32)
    lanes = lax.iota(_i32, 16)

    def zero_mv(i, carry):
        for jj in range(8):
            m_v[i, pl.ds(jj * 16, 16)] = zero16
        return carry

    lax.fori_loop(0, C4, zero_mv, 0)

    def zero_accm(k, carry):
        pltpu.sync_copy(m_v, accM.at[pl.ds(s0 + k * C4, C4)])
        return carry

    lax.fori_loop(0, SROWS // C4, zero_accm, 0)
    plsc.subcore_barrier()

    col0 = jnp.zeros((16,), _i32)
    col1 = col0 + 1
    col2 = col0 + 2
    col3 = col0 + 3

    def chunk_body(t, carry):
        chunk = wid * NCHUNK4 + t
        e0 = chunk * C4
        pltpu.sync_copy(row_hbm.at[pl.ds(chunk, 1)], idx)
        pltpu.sync_copy(m0_hbm.at[pl.ds(e0, C4)], m0_v)
        pltpu.sync_copy(m1_hbm.at[pl.ds(e0, C4)], m1_v)
        pltpu.sync_copy(m2_hbm.at[pl.ds(e0, C4)], m2_v)

        def pack_body(gg, c2carry):
            base = gg * 16
            rows = base + lanes
            plsc.store_scatter(m_v, [rows, col0], m0_v[pl.ds(base, 16)])
            plsc.store_scatter(m_v, [rows, col1], m1_v[pl.ds(base, 16)])
            plsc.store_scatter(m_v, [rows, col2], m2_v[pl.ds(base, 16)])
            plsc.store_scatter(m_v, [rows, col3], ones16)
            return c2carry

        lax.fori_loop(0, C4 // 16, pack_body, 0)
        pltpu.sync_copy(m_v, accM.at[idx.at[0]], add=True)
        return carry

    lax.fori_loop(0, NCHUNK4, chunk_body, 0)
    plsc.subcore_barrier()

    def fgroup(g, carry):
        r0 = s0 + g * C4
        pltpu.sync_copy(accM.at[pl.ds(r0, C4)], m_v)
        out0 = pl.multiple_of(cid * NP + r0, C4)
        pltpu.sync_copy(m_v, pm_hbm.at[pl.ds(out0, C4)])
        return carry

    lax.fori_loop(0, SROWS // C4, fgroup, 0)


def _k4b(M0, M1, M2, row2):
    mesh = plsc.VectorSubcoreMesh(core_axis_name="c", subcore_axis_name="s")
    return pl.kernel(
        _k4b_body,
        out_type=jax.ShapeDtypeStruct((2 * NP, H), _f32),
        mesh=mesh,
        compiler_params=pltpu.CompilerParams(needs_layout_passes=False),
        scratch_types=[
            pltpu.VMEM((1, 128), _i32),
            pltpu.VMEM((C4, H), _f32),
            pltpu.VMEM((C4,), _f32),
            pltpu.VMEM((C4,), _f32),
            pltpu.VMEM((C4,), _f32),
            pltpu.VMEM_SHARED((NP, H), _f32),
            pltpu.SemaphoreType.DMA,
        ],
    )(M0, M1, M2, row2)


def _k5_body(he_ref, p0_ref, p1_ref, pm0_ref, pm1_ref,
             n1a_ref, n1b_ref, nb1_ref, n2_ref, nb2_ref,
             h_ref, v_ref):
    he = he_ref[...]
    agg = p0_ref[...] + p1_ref[...]
    t = jnp.dot(he.astype(_bf16), n1a_ref[...].astype(_bf16),
                preferred_element_type=_f32)
    t = t + jnp.dot(agg.astype(_bf16), n1b_ref[...].astype(_bf16),
                    preferred_element_type=_f32)
    t = jnp.maximum(t + nb1_ref[...], 0.0)
    h_ref[...] = he + jnp.dot(t.astype(_bf16), n2_ref[...].astype(_bf16),
                              preferred_element_type=_f32) + nb2_ref[...]

    vm = pm0_ref[...] + pm1_ref[...]
    cnt = jnp.maximum(vm[:, 3:4], 1.0)
    v1 = vm[:, 0:3] / cnt
    norm = jnp.sqrt(jnp.sum(v1 * v1, axis=1, keepdims=True))
    v3 = v1 / jnp.maximum(norm, 1e-12)
    v_ref[...] = jnp.concatenate([v3, jnp.zeros((NB, 13), _f32)], axis=1)


def _k5(he, P0, P1, PM0, PM1, n1a, n1b, nb1, n2, nb2):
    blk = pl.BlockSpec((NB, H), lambda i: (i, 0))
    sblk = pl.BlockSpec((NB, 16), lambda i: (i, 0))
    wblk = pl.BlockSpec((H, H), lambda i: (0, 0))
    vblk = pl.BlockSpec((1, H), lambda i: (0, 0))
    return pl.pallas_call(
        _k5_body,
        grid=(N // NB,),
        in_specs=[blk, blk, blk, blk, blk, wblk, wblk, vblk, wblk, vblk],
        out_specs=[blk, sblk],
        out_shape=[jax.ShapeDtypeStruct((N, H), _f32),
                   jax.ShapeDtypeStruct((N, 16), _f32)],
    )(he, P0, P1, PM0, PM1, n1a, n1b, nb1, n2, nb2)


@jax.jit
def _impl(h, x, edge_index, emb_W, emb_b, eW1, eb1, eW2, eb2,
          nW1, nb1, nW2, nb2, wW1, wb1, wW2, wb2):
    row = edge_index[0].astype(_i32)
    col = edge_index[1].astype(_i32)
    row2 = jnp.pad(row, (0, EP - E)).reshape(EP // 128, 128)
    col2 = jnp.pad(col, (0, EP - E)).reshape(EP // 128, 128)
    x0 = jnp.asarray(x[:, 0], _f32)
    x1 = jnp.asarray(x[:, 1], _f32)
    x2 = jnp.asarray(x[:, 2], _f32)

    he, A, B = _k1(h, emb_W, emb_b.reshape(1, H), eW1[:H], eW1[H:2 * H])
    G, D, C2, S2, PH = _k2(A, B, x0, x1, x2, row2, col2)
    nsh = (EP // EB, EB // 128, 128)
    F, M0, M1, M2 = _k3(
        G, D.reshape(nsh), C2.reshape(nsh), S2.reshape(nsh), PH.reshape(nsh),
        eW1[2 * H:2 * H + 1], eb1.reshape(1, H), eW2, eb2.reshape(1, H),
        wW1, wb1.reshape(1, H), wW2.reshape(1, H),
        jnp.broadcast_to(wb2.reshape(1, 1), (1, H)))
    P = _k4a(F, row2)
    P0, P1 = P[:NP], P[NP:]
    PM = _k4b(M0.reshape(EP), M1.reshape(EP), M2.reshape(EP), row2)
    PM0, PM1 = PM[:NP], PM[NP:]
    hout, vout = _k5(he, P0, P1, PM0, PM1,
                     nW1[:H], nW1[H:], nb1.reshape(1, H), nW2,
                     nb2.reshape(1, H))
    v = vout[:, :3].reshape(N, 1, 3)
    return (hout, x, v)


def kernel(h, x, edge_index, emb_W, emb_b, eW1, eb1, eW2, eb2,
           nW1, nb1, nW2, nb2, wW1, wb1, wW2, wb2):
    return _impl(h, x, edge_index, emb_W, emb_b, eW1, eb1, eW2, eb2,
                 nW1, nb1, nW2, nb2, wW1, wb1, wW2, wb2)


# speedup vs baseline: 2.7221x; 1.1725x over previous
import jax
import jax.numpy as jnp
import numpy as np
from jax import lax
from jax.experimental import pallas as pl
from jax.experimental.pallas import tpu as pltpu
from jax.experimental.pallas import tpu_sc as plsc

N = 10000
E = 320000
H = 128

NC = 2
NS = 16
NW = NC * NS
C = 256
IB = C // 128
EP = 327680
NCHUNK = EP // C // NW
NP = 10240
SROWS = NP // NS
EB = 512
NB = 1000

_f32 = jnp.float32
_i32 = jnp.int32
_bf16 = jnp.bfloat16


def _atan01(q):
    s = q * q
    return q * (0.99997726 + s * (-0.33262347 + s * (0.19354346 + s * (
        -0.11643287 + s * (0.05265332 + s * (-0.01172120))))))


def _k1_body(h_ref, embW_ref, embb_ref, w1a_ref, w1b_ref,
             he_ref, a_ref, b_ref):
    he = jnp.dot(h_ref[...].astype(_bf16), embW_ref[...].astype(_bf16),
                 preferred_element_type=_f32)
    he = he + embb_ref[...]
    he_ref[...] = he
    heb = he.astype(_bf16)
    a_ref[...] = jnp.dot(heb, w1a_ref[...].astype(_bf16),
                         preferred_element_type=_f32)
    b_ref[...] = jnp.dot(heb, w1b_ref[...].astype(_bf16),
                         preferred_element_type=_f32)


def _k1(h, emb_W, emb_b, w1a, w1b):
    blk = pl.BlockSpec((NB, H), lambda i: (i, 0))
    wblk = pl.BlockSpec((H, H), lambda i: (0, 0))
    vblk = pl.BlockSpec((1, H), lambda i: (0, 0))
    return pl.pallas_call(
        _k1_body,
        grid=(N // NB,),
        in_specs=[blk, wblk, vblk, wblk, wblk],
        out_specs=[blk, blk, blk],
        out_shape=[jax.ShapeDtypeStruct((N, H), _f32)] * 3,
    )(h, emb_W, emb_b, w1a, w1b)


C2 = 128
NCHUNK2 = EP // C2 // NW


def _k2_body(a_hbm, b_hbm, x0_hbm, x1_hbm, x2_hbm, row_hbm, col_hbm,
             g_hbm, d_hbm, c2_hbm, s2_hbm, ph_hbm,
             idxr0, idxr1, idxc0, idxc1, av0, av1, bv0, bv1,
             x0_v, x1_v, x2_v, dv0, dv1, cv0, cv1, sv0, sv1, pv0, pv1,
             si0, si1, so0, so1):
    wid = lax.axis_index("s") * NC + lax.axis_index("c")
    idxr = [idxr0, idxr1]
    idxc = [idxc0, idxc1]
    av = [av0, av1]
    bv = [bv0, bv1]
    dv = [dv0, dv1]
    cv = [cv0, cv1]
    sv = [sv0, sv1]
    pv = [pv0, pv1]
    si = [si0, si1]
    so = [so0, so1]

    pltpu.sync_copy(x0_hbm, x0_v)
    pltpu.sync_copy(x1_hbm, x1_v)
    pltpu.sync_copy(x2_hbm, x2_v)
    lanes = lax.iota(_i32, 16)
    del lanes

    def stage_in(k, bi):
        ch = wid * NCHUNK2 + k
        pltpu.sync_copy(row_hbm.at[pl.ds(ch, 1)], idxr[bi])
        pltpu.sync_copy(col_hbm.at[pl.ds(ch, 1)], idxc[bi])
        pltpu.async_copy(a_hbm.at[idxr[bi].at[0]], av[bi], si[bi])
        pltpu.async_copy(b_hbm.at[idxc[bi].at[0]], bv[bi], si[bi])

    def drain_in(bi):
        pltpu.make_async_copy(a_hbm.at[idxr[bi].at[0]], av[bi], si[bi]).wait()
        pltpu.make_async_copy(b_hbm.at[idxc[bi].at[0]], bv[bi], si[bi]).wait()

    def compute(bi):
        def geo_body(gg, carry):
            off = gg * 16
            ir = idxr[bi][0, pl.ds(off, 16)]
            ic = idxc[bi][0, pl.ds(off, 16)]
            xd = plsc.load_gather(x0_v, [ir]) - plsc.load_gather(x0_v, [ic])
            yd = plsc.load_gather(x1_v, [ir]) - plsc.load_gather(x1_v, [ic])
            zd = plsc.load_gather(x2_v, [ir]) - plsc.load_gather(x2_v, [ic])
            x2 = xd * xd
            y2 = yd * yd
            rxy2 = x2 + y2
            d = rxy2 + zd * zd
            safe = jnp.maximum(rxy2, 1e-30)
            pos = rxy2 > 0.0
            c2 = jnp.where(pos, (x2 - y2) / safe, 1.0)
            s2 = jnp.where(pos, (2.0 * xd * yd) / safe, 0.0)
            ai = plsc.bitcast(safe, _i32)
            y = plsc.bitcast(_i32(0x5F3759DF) - (ai >> 1), _f32)
            y = y * (1.5 - 0.5 * safe * y * y)
            y = y * (1.5 - 0.5 * safe * y * y)
            y = y * (1.5 - 0.5 * safe * y * y)
            rxy = safe * y
            az = jnp.abs(zd)
            mx = jnp.maximum(az, rxy)
            mn = jnp.minimum(az, rxy)
            q = mn / jnp.maximum(mx, 1e-30)
            aa = _atan01(q)
            phi = jnp.sign(zd) * jnp.where(az > rxy,
                                           np.float32(np.pi / 2) - aa, aa)
            dv[bi][pl.ds(off, 16)] = d
            cv[bi][pl.ds(off, 16)] = c2
            sv[bi][pl.ds(off, 16)] = s2
            pv[bi][pl.ds(off, 16)] = phi
            return carry

        lax.fori_loop(0, C2 // 16, geo_body, 0)

        def add_body(i, carry):
            for jj in range(8):
                sl = pl.ds(jj * 16, 16)
                av[bi][i, sl] = av[bi][i, sl] + bv[bi][i, sl]
            return carry

        lax.fori_loop(0, C2, add_body, 0)

    def issue_outs(k, bi):
        e0 = (wid * NCHUNK2 + k) * C2
        pltpu.async_copy(av[bi], g_hbm.at[pl.ds(e0, C2)], so[bi])
        pltpu.async_copy(dv[bi], d_hbm.at[pl.ds(e0, C2)], so[bi])
        pltpu.async_copy(cv[bi], c2_hbm.at[pl.ds(e0, C2)], so[bi])
        pltpu.async_copy(sv[bi], s2_hbm.at[pl.ds(e0, C2)], so[bi])
        pltpu.async_copy(pv[bi], ph_hbm.at[pl.ds(e0, C2)], so[bi])

    def drain_out(bi):
        pltpu.make_async_copy(av[bi], g_hbm.at[pl.ds(0, C2)], so[bi]).wait()
        pltpu.make_async_copy(dv[bi], d_hbm.at[pl.ds(0, C2)], so[bi]).wait()
        pltpu.make_async_copy(cv[bi], c2_hbm.at[pl.ds(0, C2)], so[bi]).wait()
        pltpu.make_async_copy(sv[bi], s2_hbm.at[pl.ds(0, C2)], so[bi]).wait()
        pltpu.make_async_copy(pv[bi], ph_hbm.at[pl.ds(0, C2)], so[bi]).wait()

    stage_in(0, 0)
    drain_in(0)
    stage_in(1, 1)
    compute(0)
    issue_outs(0, 0)
    drain_in(1)
    drain_out(0)
    stage_in(2, 0)
    compute(1)
    issue_outs(1, 1)

    def interior(tt, carry):
        k0 = 2 * tt
        drain_in(0)
        drain_out(1)
        stage_in(k0 + 1, 1)
        compute(0)
        issue_outs(k0, 0)
        drain_in(1)
        drain_out(0)
        stage_in(k0 + 2, 0)
        compute(1)
        issue_outs(k0 + 1, 1)
        return carry

    lax.fori_loop(1, NCHUNK2 // 2 - 1, interior, 0)
    drain_in(0)
    drain_out(1)
    stage_in(NCHUNK2 - 1, 1)
    compute(0)
    issue_outs(NCHUNK2 - 2, 0)
    drain_in(1)
    compute(1)
    issue_outs(NCHUNK2 - 1, 1)
    drain_out(0)
    drain_out(1)


def _k2(A, B, x0, x1, x2, row2, col2):
    mesh = plsc.VectorSubcoreMesh(core_axis_name="c", subcore_axis_name="s")
    return pl.kernel(
        _k2_body,
        out_type=[jax.ShapeDtypeStruct((EP, H), _f32),
                  jax.ShapeDtypeStruct((EP,), _f32),
                  jax.ShapeDtypeStruct((EP,), _f32),
                  jax.ShapeDtypeStruct((EP,), _f32),
                  jax.ShapeDtypeStruct((EP,), _f32)],
        mesh=mesh,
        compiler_params=pltpu.CompilerParams(needs_layout_passes=False),
        scratch_types=[
            pltpu.VMEM((1, 128), _i32),
            pltpu.VMEM((1, 128), _i32),
            pltpu.VMEM((1, 128), _i32),
            pltpu.VMEM((1, 128), _i32),
            pltpu.VMEM((C2, H), _f32),
            pltpu.VMEM((C2, H), _f32),
            pltpu.VMEM((C2, H), _f32),
            pltpu.VMEM((C2, H), _f32),
            pltpu.VMEM((N,), _f32),
            pltpu.VMEM((N,), _f32),
            pltpu.VMEM((N,), _f32),
            pltpu.VMEM((C2,), _f32),
            pltpu.VMEM((C2,), _f32),
            pltpu.VMEM((C2,), _f32),
            pltpu.VMEM((C2,), _f32),
            pltpu.VMEM((C2,), _f32),
            pltpu.VMEM((C2,), _f32),
            pltpu.VMEM((C2,), _f32),
            pltpu.VMEM((C2,), _f32),
            pltpu.SemaphoreType.DMA,
            pltpu.SemaphoreType.DMA,
            pltpu.SemaphoreType.DMA,
            pltpu.SemaphoreType.DMA,
        ],
    )(A, B, x0, x1, x2, row2, col2)


def _k3_body(g_ref, d_ref, c2_ref, s2_ref, ph_ref, u_ref, b1_ref,
             w2_ref, b2_ref, ww1_ref, wb1_ref, wv2_ref, wb2_ref,
             f_ref, m0_ref, m1_ref, m2_ref):
    ident = (lax.broadcasted_iota(_i32, (128, 128), 0) ==
             lax.broadcasted_iota(_i32, (128, 128), 1)).astype(_f32)

    def unpack_col(ref):
        pk = jnp.reshape(ref[...], (EB // 128, 128))
        cols = [jnp.sum(jnp.broadcast_to(pk[i:i + 1, :], (128, 128)) * ident,
                        axis=1, keepdims=True) for i in range(EB // 128)]
        return jnp.concatenate(cols, axis=0)

    def pack_row(colv):
        rows = [jnp.sum(colv[i * 128:(i + 1) * 128] * ident,
                        axis=0, keepdims=True) for i in range(EB // 128)]
        return jnp.reshape(jnp.concatenate(rows, axis=0), (1, EB // 128, 128))

    d = unpack_col(d_ref)
    c2 = unpack_col(c2_ref)
    s2 = unpack_col(s2_ref)
    ph = unpack_col(ph_ref)

    db = d.astype(_bf16).astype(_f32)
    ub = u_ref[...].astype(_bf16).astype(_f32)
    pre = g_ref[...] + db * ub + b1_ref[...]
    t1 = jnp.maximum(pre, 0.0)
    F0 = jnp.dot(t1.astype(_bf16), w2_ref[...].astype(_bf16),
                 preferred_element_type=_f32) + b2_ref[...]
    t2 = jnp.maximum(
        jnp.dot(F0.astype(_bf16), ww1_ref[...].astype(_bf16),
                preferred_element_type=_f32) + wb1_ref[...],
        0.0)
    t2b = t2.astype(_bf16).astype(_f32)
    wv2b = wv2_ref[...].astype(_bf16).astype(_f32)
    w = jnp.sum(t2b * wv2b, axis=1, keepdims=True) + wb2_ref[:, 0:1]

    eids = pl.program_id(0) * EB + lax.broadcasted_iota(_i32, (EB, 1), 0)
    valid = eids < E
    f_ref[...] = jnp.where(valid, F0, 0.0)
    dw = jnp.where(valid, d * w, 0.0)
    m0_ref[...] = pack_row(dw * c2)
    m1_ref[...] = pack_row(dw * s2)
    m2_ref[...] = pack_row(dw * ph)


def _k3(G, D2, C22, S22, PH2, u, b1, eW2, b2, wW1, wb1, wv2, wb2):
    eblk = pl.BlockSpec((EB, H), lambda i: (i, 0))
    nblk = pl.BlockSpec((1, EB // 128, 128), lambda i: (i, 0, 0))
    wblk = pl.BlockSpec((H, H), lambda i: (0, 0))
    vblk = pl.BlockSpec((1, H), lambda i: (0, 0))
    nsh = jax.ShapeDtypeStruct((EP // EB, EB // 128, 128), _f32)
    return pl.pallas_call(
        _k3_body,
        grid=(EP // EB,),
        in_specs=[eblk, nblk, nblk, nblk, nblk,
                  vblk, vblk, wblk, vblk, wblk, vblk, vblk, vblk],
        out_specs=[eblk, nblk, nblk, nblk],
        out_shape=[jax.ShapeDtypeStruct((EP, H), _f32), nsh, nsh, nsh],
    )(G, D2, C22, S22, PH2, u, b1, eW2, b2, wW1, wb1, wv2, wb2)


C4 = 128
NCHUNK4 = EP // C4 // NW


def _k4a_body(f_hbm, row_hbm, p_hbm, idx, f_v, accF, sem):
    cid = lax.axis_index("c")
    sid = lax.axis_index("s")
    wid = sid * NC + cid
    s0 = sid * SROWS

    zero16 = jnp.zeros((16,), _f32)

    def zero_fv(i, carry):
        for jj in range(8):
            f_v[i, pl.ds(jj * 16, 16)] = zero16
        return carry

    lax.fori_loop(0, C4, zero_fv, 0)

    def zero_accf(k, carry):
        pltpu.sync_copy(f_v.at[pl.ds(0, SROWS // 8)],
                        accF.at[pl.ds(s0 + k * (SROWS // 8), SROWS // 8)])
        return carry

    lax.fori_loop(0, 8, zero_accf, 0)
    plsc.subcore_barrier()

    def chunk_body(t, carry):
        chunk = wid * NCHUNK4 + t
        e0 = chunk * C4
        pltpu.sync_copy(row_hbm.at[pl.ds(chunk, 1)], idx)
        pltpu.sync_copy(f_hbm.at[pl.ds(e0, C4)], f_v)
        pltpu.sync_copy(f_v, accF.at[idx.at[0]], add=True)
        return carry

    lax.fori_loop(0, NCHUNK4, chunk_body, 0)
    plsc.subcore_barrier()

    def fgroup(g, carry):
        r0 = s0 + g * C4
        pltpu.sync_copy(accF.at[pl.ds(r0, C4)], f_v)
        out0 = pl.multiple_of(cid * NP + r0, C4)
        pltpu.sync_copy(f_v, p_hbm.at[pl.ds(out0, C4)])
        return carry

    lax.fori_loop(0, SROWS // C4, fgroup, 0)


def _k4a(F, row2):
    mesh = plsc.VectorSubcoreMesh(core_axis_name="c", subcore_axis_name="s")
    return pl.kernel(
        _k4a_body,
        out_type=jax.ShapeDtypeStruct((2 * NP, H), _f32),
        mesh=mesh,
        compiler_params=pltpu.CompilerParams(needs_layout_passes=False),
        scratch_types=[
            pltpu.VMEM((1, 128), _i32),
            pltpu.VMEM((C4, H), _f32),
            pltpu.VMEM_SHARED((NP, H), _f32),
            pltpu.SemaphoreType.DMA,
        ],
    )(F, row2)


def _k4b_body(m0_hbm, m1_hbm, m2_hbm, row_hbm, pm_hbm,
              idx, m_v, m0_v, m1_v, m2_v, accM, sem):
    cid = lax.axis_index("c")
    sid = lax.axis_index("s")
    wid = sid * NC + cid
    s0 = sid * SROWS

    zero16 = jnp.zeros((16,), _f32)
    ones16 = jnp.ones((16,), _f32)
    lanes = lax.iota(_i32, 16)

    def zero_mv(i, carry):
        for jj in range(8):
            m_v[i, pl.ds(jj * 16, 16)] = zero16
        return carry

    lax.fori_loop(0, C4, zero_mv, 0)

    def zero_accm(k, carry):
        pltpu.sync_copy(m_v, accM.at[pl.ds(s0 + k * C4, C4)])
        return carry

    lax.fori_loop(0, SROWS // C4, zero_accm, 0)
    plsc.subcore_barrier()

    col0 = jnp.zeros((16,), _i32)
    col1 = col0 + 1
    col2 = col0 + 2
    col3 = col0 + 3

    def chunk_body(t, carry):
        chunk = wid * NCHUNK4 + t
        e0 = chunk * C4
        pltpu.sync_copy(row_hbm.at[pl.ds(chunk, 1)], idx)
        pltpu.sync_copy(m0_hbm.at[pl.ds(e0, C4)], m0_v)
        pltpu.sync_copy(m1_hbm.at[pl.ds(e0, C4)], m1_v)
        pltpu.sync_copy(m2_hbm.at[pl.ds(e0, C4)], m2_v)

        def pack_body(gg, c2carry):
            base = gg * 16
            rows = base + lanes
            plsc.store_scatter(m_v, [rows, col0], m0_v[pl.ds(base, 16)])
            plsc.store_scatter(m_v, [rows, col1], m1_v[pl.ds(base, 16)])
            plsc.store_scatter(m_v, [rows, col2], m2_v[pl.ds(base, 16)])
            plsc.store_scatter(m_v, [rows, col3], ones16)
            return c2carry

        lax.fori_loop(0, C4 // 16, pack_body, 0)
        pltpu.sync_copy(m_v, accM.at[idx.at[0]], add=True)
        return carry

    lax.fori_loop(0, NCHUNK4, chunk_body, 0)
    plsc.subcore_barrier()

    def fgroup(g, carry):
        r0 = s0 + g * C4
        pltpu.sync_copy(accM.at[pl.ds(r0, C4)], m_v)
        out0 = pl.multiple_of(cid * NP + r0, C4)
        pltpu.sync_copy(m_v, pm_hbm.at[pl.ds(out0, C4)])
        return carry

    lax.fori_loop(0, SROWS // C4, fgroup, 0)


def _k4b(M0, M1, M2, row2):
    mesh = plsc.VectorSubcoreMesh(core_axis_name="c", subcore_axis_name="s")
    return pl.kernel(
        _k4b_body,
        out_type=jax.ShapeDtypeStruct((2 * NP, H), _f32),
        mesh=mesh,
        compiler_params=pltpu.CompilerParams(needs_layout_passes=False),
        scratch_types=[
            pltpu.VMEM((1, 128), _i32),
            pltpu.VMEM((C4, H), _f32),
            pltpu.VMEM((C4,), _f32),
            pltpu.VMEM((C4,), _f32),
            pltpu.VMEM((C4,), _f32),
            pltpu.VMEM_SHARED((NP, H), _f32),
            pltpu.SemaphoreType.DMA,
        ],
    )(M0, M1, M2, row2)


def _k5_body(he_ref, p0_ref, p1_ref, pm0_ref, pm1_ref,
             n1a_ref, n1b_ref, nb1_ref, n2_ref, nb2_ref,
             h_ref, v_ref):
    he = he_ref[...]
    agg = p0_ref[...] + p1_ref[...]
    t = jnp.dot(he.astype(_bf16), n1a_ref[...].astype(_bf16),
                preferred_element_type=_f32)
    t = t + jnp.dot(agg.astype(_bf16), n1b_ref[...].astype(_bf16),
                    preferred_element_type=_f32)
    t = jnp.maximum(t + nb1_ref[...], 0.0)
    h_ref[...] = he + jnp.dot(t.astype(_bf16), n2_ref[...].astype(_bf16),
                              preferred_element_type=_f32) + nb2_ref[...]

    vm = pm0_ref[...] + pm1_ref[...]
    cnt = jnp.maximum(vm[:, 3:4], 1.0)
    v1 = vm[:, 0:3] / cnt
    norm = jnp.sqrt(jnp.sum(v1 * v1, axis=1, keepdims=True))
    v3 = v1 / jnp.maximum(norm, 1e-12)
    v_ref[...] = jnp.concatenate([v3, jnp.zeros((NB, 13), _f32)], axis=1)


def _k5(he, P0, P1, PM0, PM1, n1a, n1b, nb1, n2, nb2):
    blk = pl.BlockSpec((NB, H), lambda i: (i, 0))
    sblk = pl.BlockSpec((NB, 16), lambda i: (i, 0))
    wblk = pl.BlockSpec((H, H), lambda i: (0, 0))
    vblk = pl.BlockSpec((1, H), lambda i: (0, 0))
    return pl.pallas_call(
        _k5_body,
        grid=(N // NB,),
        in_specs=[blk, blk, blk, blk, blk, wblk, wblk, vblk, wblk, vblk],
        out_specs=[blk, sblk],
        out_shape=[jax.ShapeDtypeStruct((N, H), _f32),
                   jax.ShapeDtypeStruct((N, 16), _f32)],
    )(he, P0, P1, PM0, PM1, n1a, n1b, nb1, n2, nb2)


@jax.jit
def _impl(h, x, edge_index, emb_W, emb_b, eW1, eb1, eW2, eb2,
          nW1, nb1, nW2, nb2, wW1, wb1, wW2, wb2):
    row = edge_index[0].astype(_i32)
    col = edge_index[1].astype(_i32)
    row2 = jnp.pad(row, (0, EP - E)).reshape(EP // 128, 128)
    col2 = jnp.pad(col, (0, EP - E)).reshape(EP // 128, 128)
    x0 = jnp.asarray(x[:, 0], _f32)
    x1 = jnp.asarray(x[:, 1], _f32)
    x2 = jnp.asarray(x[:, 2], _f32)

    he, A, B = _k1(h, emb_W, emb_b.reshape(1, H), eW1[:H], eW1[H:2 * H])
    G, D, C2, S2, PH = _k2(A, B, x0, x1, x2, row2, col2)
    nsh = (EP // EB, EB // 128, 128)
    F, M0, M1, M2 = _k3(
        G, D.reshape(nsh), C2.reshape(nsh), S2.reshape(nsh), PH.reshape(nsh),
        eW1[2 * H:2 * H + 1], eb1.reshape(1, H), eW2, eb2.reshape(1, H),
        wW1, wb1.reshape(1, H), wW2.reshape(1, H),
        jnp.broadcast_to(wb2.reshape(1, 1), (1, H)))
    P = _k4a(F, row2)
    P0, P1 = P[:NP], P[NP:]
    PM = _k4b(M0.reshape(EP), M1.reshape(EP), M2.reshape(EP), row2)
    PM0, PM1 = PM[:NP], PM[NP:]
    hout, vout = _k5(he, P0, P1, PM0, PM1,
                     nW1[:H], nW1[H:], nb1.reshape(1, H), nW2,
                     nb2.reshape(1, H))
    v = vout[:, :3].reshape(N, 1, 3)
    return (hout, x, v)


def kernel(h, x, edge_index, emb_W, emb_b, eW1, eb1, eW2, eb2,
           nW1, nb1, nW2, nb2, wW1, wb1, wW2, wb2):
    return _impl(h, x, edge_index, emb_W, emb_b, eW1, eb1, eW2, eb2,
                 nW1, nb1, nW2, nb2, wW1, wb1, wW2, wb2)


# speedup vs baseline: 3.0863x; 1.1338x over previous
import jax
import jax.numpy as jnp
import numpy as np
from jax import lax
from jax.experimental import pallas as pl
from jax.experimental.pallas import tpu as pltpu
from jax.experimental.pallas import tpu_sc as plsc

N = 10000
E = 320000
H = 128

NC = 2
NS = 16
NW = NC * NS
C = 256
IB = C // 128
EP = 327680
NCHUNK = EP // C // NW
NP = 10240
SROWS = NP // NS
EB = 512
NB = 1000

_f32 = jnp.float32
_i32 = jnp.int32
_bf16 = jnp.bfloat16


def _atan01(q):
    s = q * q
    return q * (0.99997726 + s * (-0.33262347 + s * (0.19354346 + s * (
        -0.11643287 + s * (0.05265332 + s * (-0.01172120))))))


def _k1_body(h_ref, embW_ref, embb_ref, w1a_ref, w1b_ref,
             he_ref, a_ref, b_ref):
    he = jnp.dot(h_ref[...].astype(_bf16), embW_ref[...].astype(_bf16),
                 preferred_element_type=_f32)
    he = he + embb_ref[...]
    he_ref[...] = he
    heb = he.astype(_bf16)
    a_ref[...] = jnp.dot(heb, w1a_ref[...].astype(_bf16),
                         preferred_element_type=_f32)
    b_ref[...] = jnp.dot(heb, w1b_ref[...].astype(_bf16),
                         preferred_element_type=_f32)


def _k1(h, emb_W, emb_b, w1a, w1b):
    blk = pl.BlockSpec((NB, H), lambda i: (i, 0))
    wblk = pl.BlockSpec((H, H), lambda i: (0, 0))
    vblk = pl.BlockSpec((1, H), lambda i: (0, 0))
    return pl.pallas_call(
        _k1_body,
        grid=(N // NB,),
        in_specs=[blk, wblk, vblk, wblk, wblk],
        out_specs=[blk, blk, blk],
        out_shape=[jax.ShapeDtypeStruct((N, H), _f32)] * 3,
    )(h, emb_W, emb_b, w1a, w1b)


C2 = 128
NCHUNK2 = EP // C2 // NW


def _k2_body(a_hbm, b_hbm, x0_hbm, x1_hbm, x2_hbm, row_hbm, col_hbm,
             g_hbm, d_hbm, c2_hbm, s2_hbm, ph_hbm,
             idxr0, idxr1, idxc0, idxc1, av0, av1, bv0, bv1,
             x0_v, x1_v, x2_v, dv0, dv1, cv0, cv1, sv0, sv1, pv0, pv1,
             si0, si1, so0, so1):
    wid = lax.axis_index("s") * NC + lax.axis_index("c")
    idxr = [idxr0, idxr1]
    idxc = [idxc0, idxc1]
    av = [av0, av1]
    bv = [bv0, bv1]
    dv = [dv0, dv1]
    cv = [cv0, cv1]
    sv = [sv0, sv1]
    pv = [pv0, pv1]
    si = [si0, si1]
    so = [so0, so1]

    pltpu.sync_copy(x0_hbm, x0_v)
    pltpu.sync_copy(x1_hbm, x1_v)
    pltpu.sync_copy(x2_hbm, x2_v)
    lanes = lax.iota(_i32, 16)
    del lanes

    def stage_in(k, bi):
        ch = wid * NCHUNK2 + k
        pltpu.sync_copy(row_hbm.at[pl.ds(ch, 1)], idxr[bi])
        pltpu.sync_copy(col_hbm.at[pl.ds(ch, 1)], idxc[bi])
        pltpu.async_copy(a_hbm.at[idxr[bi].at[0]], av[bi], si[bi])
        pltpu.async_copy(b_hbm.at[idxc[bi].at[0]], bv[bi], si[bi])

    def drain_in(bi):
        pltpu.make_async_copy(a_hbm.at[idxr[bi].at[0]], av[bi], si[bi]).wait()
        pltpu.make_async_copy(b_hbm.at[idxc[bi].at[0]], bv[bi], si[bi]).wait()

    def compute(bi):
        def geo_body(gg, carry):
            off = gg * 16
            ir = idxr[bi][0, pl.ds(off, 16)]
            ic = idxc[bi][0, pl.ds(off, 16)]
            xd = plsc.load_gather(x0_v, [ir]) - plsc.load_gather(x0_v, [ic])
            yd = plsc.load_gather(x1_v, [ir]) - plsc.load_gather(x1_v, [ic])
            zd = plsc.load_gather(x2_v, [ir]) - plsc.load_gather(x2_v, [ic])
            x2 = xd * xd
            y2 = yd * yd
            rxy2 = x2 + y2
            d = rxy2 + zd * zd
            safe = jnp.maximum(rxy2, 1e-30)
            pos = rxy2 > 0.0
            c2 = jnp.where(pos, (x2 - y2) / safe, 1.0)
            s2 = jnp.where(pos, (2.0 * xd * yd) / safe, 0.0)
            ai = plsc.bitcast(safe, _i32)
            y = plsc.bitcast(_i32(0x5F3759DF) - (ai >> 1), _f32)
            y = y * (1.5 - 0.5 * safe * y * y)
            y = y * (1.5 - 0.5 * safe * y * y)
            y = y * (1.5 - 0.5 * safe * y * y)
            rxy = safe * y
            az = jnp.abs(zd)
            mx = jnp.maximum(az, rxy)
            mn = jnp.minimum(az, rxy)
            q = mn / jnp.maximum(mx, 1e-30)
            aa = _atan01(q)
            phi = jnp.sign(zd) * jnp.where(az > rxy,
                                           np.float32(np.pi / 2) - aa, aa)
            dv[bi][pl.ds(off, 16)] = d
            cv[bi][pl.ds(off, 16)] = c2
            sv[bi][pl.ds(off, 16)] = s2
            pv[bi][pl.ds(off, 16)] = phi
            return carry

        lax.fori_loop(0, C2 // 16, geo_body, 0)

        def add_body(i, carry):
            for jj in range(8):
                sl = pl.ds(jj * 16, 16)
                av[bi][i, sl] = av[bi][i, sl] + bv[bi][i, sl]
            return carry

        lax.fori_loop(0, C2, add_body, 0)

    def issue_outs(k, bi):
        e0 = (wid * NCHUNK2 + k) * C2
        pltpu.async_copy(av[bi], g_hbm.at[pl.ds(e0, C2)], so[bi])
        pltpu.async_copy(dv[bi], d_hbm.at[pl.ds(e0, C2)], so[bi])
        pltpu.async_copy(cv[bi], c2_hbm.at[pl.ds(e0, C2)], so[bi])
        pltpu.async_copy(sv[bi], s2_hbm.at[pl.ds(e0, C2)], so[bi])
        pltpu.async_copy(pv[bi], ph_hbm.at[pl.ds(e0, C2)], so[bi])

    def drain_out(bi):
        pltpu.make_async_copy(av[bi], g_hbm.at[pl.ds(0, C2)], so[bi]).wait()
        pltpu.make_async_copy(dv[bi], d_hbm.at[pl.ds(0, C2)], so[bi]).wait()
        pltpu.make_async_copy(cv[bi], c2_hbm.at[pl.ds(0, C2)], so[bi]).wait()
        pltpu.make_async_copy(sv[bi], s2_hbm.at[pl.ds(0, C2)], so[bi]).wait()
        pltpu.make_async_copy(pv[bi], ph_hbm.at[pl.ds(0, C2)], so[bi]).wait()

    stage_in(0, 0)
    drain_in(0)
    stage_in(1, 1)
    compute(0)
    issue_outs(0, 0)
    drain_in(1)
    drain_out(0)
    stage_in(2, 0)
    compute(1)
    issue_outs(1, 1)

    def interior(tt, carry):
        k0 = 2 * tt
        drain_in(0)
        drain_out(1)
        stage_in(k0 + 1, 1)
        compute(0)
        issue_outs(k0, 0)
        drain_in(1)
        drain_out(0)
        stage_in(k0 + 2, 0)
        compute(1)
        issue_outs(k0 + 1, 1)
        return carry

    lax.fori_loop(1, NCHUNK2 // 2 - 1, interior, 0)
    drain_in(0)
    drain_out(1)
    stage_in(NCHUNK2 - 1, 1)
    compute(0)
    issue_outs(NCHUNK2 - 2, 0)
    drain_in(1)
    compute(1)
    issue_outs(NCHUNK2 - 1, 1)
    drain_out(0)
    drain_out(1)


def _k2(A, B, x0, x1, x2, row2, col2):
    mesh = plsc.VectorSubcoreMesh(core_axis_name="c", subcore_axis_name="s")
    return pl.kernel(
        _k2_body,
        out_type=[jax.ShapeDtypeStruct((EP, H), _f32),
                  jax.ShapeDtypeStruct((EP,), _f32),
                  jax.ShapeDtypeStruct((EP,), _f32),
                  jax.ShapeDtypeStruct((EP,), _f32),
                  jax.ShapeDtypeStruct((EP,), _f32)],
        mesh=mesh,
        compiler_params=pltpu.CompilerParams(needs_layout_passes=False),
        scratch_types=[
            pltpu.VMEM((1, 128), _i32),
            pltpu.VMEM((1, 128), _i32),
            pltpu.VMEM((1, 128), _i32),
            pltpu.VMEM((1, 128), _i32),
            pltpu.VMEM((C2, H), _f32),
            pltpu.VMEM((C2, H), _f32),
            pltpu.VMEM((C2, H), _f32),
            pltpu.VMEM((C2, H), _f32),
            pltpu.VMEM((N,), _f32),
            pltpu.VMEM((N,), _f32),
            pltpu.VMEM((N,), _f32),
            pltpu.VMEM((C2,), _f32),
            pltpu.VMEM((C2,), _f32),
            pltpu.VMEM((C2,), _f32),
            pltpu.VMEM((C2,), _f32),
            pltpu.VMEM((C2,), _f32),
            pltpu.VMEM((C2,), _f32),
            pltpu.VMEM((C2,), _f32),
            pltpu.VMEM((C2,), _f32),
            pltpu.SemaphoreType.DMA,
            pltpu.SemaphoreType.DMA,
            pltpu.SemaphoreType.DMA,
            pltpu.SemaphoreType.DMA,
        ],
    )(A, B, x0, x1, x2, row2, col2)


def _k3_body(g_ref, d_ref, c2_ref, s2_ref, ph_ref, u_ref, b1_ref,
             w2_ref, b2_ref, ww1_ref, wb1_ref, wv2_ref, wb2_ref,
             f_ref, m0_ref, m1_ref, m2_ref):
    ident = (lax.broadcasted_iota(_i32, (128, 128), 0) ==
             lax.broadcasted_iota(_i32, (128, 128), 1)).astype(_f32)

    def unpack_col(ref):
        pk = jnp.reshape(ref[...], (EB // 128, 128))
        cols = [jnp.sum(jnp.broadcast_to(pk[i:i + 1, :], (128, 128)) * ident,
                        axis=1, keepdims=True) for i in range(EB // 128)]
        return jnp.concatenate(cols, axis=0)

    def pack_row(colv):
        rows = [jnp.sum(colv[i * 128:(i + 1) * 128] * ident,
                        axis=0, keepdims=True) for i in range(EB // 128)]
        return jnp.reshape(jnp.concatenate(rows, axis=0), (1, EB // 128, 128))

    d = unpack_col(d_ref)
    c2 = unpack_col(c2_ref)
    s2 = unpack_col(s2_ref)
    ph = unpack_col(ph_ref)

    db = d.astype(_bf16).astype(_f32)
    ub = u_ref[...].astype(_bf16).astype(_f32)
    pre = g_ref[...] + db * ub + b1_ref[...]
    t1 = jnp.maximum(pre, 0.0)
    F0 = jnp.dot(t1.astype(_bf16), w2_ref[...].astype(_bf16),
                 preferred_element_type=_f32) + b2_ref[...]
    t2 = jnp.maximum(
        jnp.dot(F0.astype(_bf16), ww1_ref[...].astype(_bf16),
                preferred_element_type=_f32) + wb1_ref[...],
        0.0)
    t2b = t2.astype(_bf16).astype(_f32)
    wv2b = wv2_ref[...].astype(_bf16).astype(_f32)
    w = jnp.sum(t2b * wv2b, axis=1, keepdims=True) + wb2_ref[:, 0:1]

    eids = pl.program_id(0) * EB + lax.broadcasted_iota(_i32, (EB, 1), 0)
    valid = eids < E
    f_ref[...] = jnp.where(valid, F0, 0.0)
    dw = jnp.where(valid, d * w, 0.0)
    m0_ref[...] = pack_row(dw * c2)
    m1_ref[...] = pack_row(dw * s2)
    m2_ref[...] = pack_row(dw * ph)


def _k3(G, D2, C22, S22, PH2, u, b1, eW2, b2, wW1, wb1, wv2, wb2):
    eblk = pl.BlockSpec((EB, H), lambda i: (i, 0))
    nblk = pl.BlockSpec((1, EB // 128, 128), lambda i: (i, 0, 0))
    wblk = pl.BlockSpec((H, H), lambda i: (0, 0))
    vblk = pl.BlockSpec((1, H), lambda i: (0, 0))
    nsh = jax.ShapeDtypeStruct((EP // EB, EB // 128, 128), _f32)
    return pl.pallas_call(
        _k3_body,
        grid=(EP // EB,),
        in_specs=[eblk, nblk, nblk, nblk, nblk,
                  vblk, vblk, wblk, vblk, wblk, vblk, vblk, vblk],
        out_specs=[eblk, nblk, nblk, nblk],
        out_shape=[jax.ShapeDtypeStruct((EP, H), _f32), nsh, nsh, nsh],
    )(G, D2, C22, S22, PH2, u, b1, eW2, b2, wW1, wb1, wv2, wb2)


C4 = 128
NCHUNK4 = EP // C4 // NW


def _k4a_body(f_hbm, row_hbm, p_hbm, idx0, idx1, fv0, fv1, accF,
              sl0, sl1, ss0, ss1):
    cid = lax.axis_index("c")
    sid = lax.axis_index("s")
    wid = sid * NC + cid
    s0 = sid * SROWS
    idx = [idx0, idx1]
    fv = [fv0, fv1]
    sl = [sl0, sl1]
    ss = [ss0, ss1]

    zero16 = jnp.zeros((16,), _f32)

    def zero_fv(i, carry):
        for jj in range(8):
            fv0[i, pl.ds(jj * 16, 16)] = zero16
        return carry

    lax.fori_loop(0, C4, zero_fv, 0)

    def zero_accf(k, carry):
        pltpu.sync_copy(fv0.at[pl.ds(0, SROWS // 8)],
                        accF.at[pl.ds(s0 + k * (SROWS // 8), SROWS // 8)])
        return carry

    lax.fori_loop(0, 8, zero_accf, 0)
    plsc.subcore_barrier()

    def stage(k, bi):
        ch = wid * NCHUNK4 + k
        pltpu.async_copy(row_hbm.at[pl.ds(ch, 1)], idx[bi], sl[bi])
        pltpu.async_copy(f_hbm.at[pl.ds(ch * C4, C4)], fv[bi], sl[bi])

    def wait_load(bi):
        pltpu.make_async_copy(row_hbm.at[pl.ds(0, 1)], idx[bi], sl[bi]).wait()
        pltpu.make_async_copy(f_hbm.at[pl.ds(0, C4)], fv[bi], sl[bi]).wait()

    def scatter(bi):
        pltpu.async_copy(fv[bi], accF.at[idx[bi].at[0]], ss[bi], add=True)

    def drain_scatter(bi):
        pltpu.make_async_copy(fv[bi], accF.at[idx[bi].at[0]], ss[bi]).wait()

    stage(0, 0)
    wait_load(0)
    stage(1, 1)
    scatter(0)

    def interior(tt, carry):
        k = 1 + 2 * tt
        wait_load(1)
        drain_scatter(0)
        stage(k + 1, 0)
        scatter(1)
        wait_load(0)
        drain_scatter(1)
        stage(k + 2, 1)
        scatter(0)
        return carry

    lax.fori_loop(0, (NCHUNK4 - 2) // 2, interior, 0)
    wait_load(1)
    scatter(1)
    drain_scatter(0)
    drain_scatter(1)
    plsc.subcore_barrier()

    def fgroup(g, carry):
        r0 = s0 + g * C4
        pltpu.sync_copy(accF.at[pl.ds(r0, C4)], fv0)
        out0 = pl.multiple_of(cid * NP + r0, C4)
        pltpu.sync_copy(fv0, p_hbm.at[pl.ds(out0, C4)])
        return carry

    lax.fori_loop(0, SROWS // C4, fgroup, 0)


def _k4a(F, row2):
    mesh = plsc.VectorSubcoreMesh(core_axis_name="c", subcore_axis_name="s")
    return pl.kernel(
        _k4a_body,
        out_type=jax.ShapeDtypeStruct((2 * NP, H), _f32),
        mesh=mesh,
        compiler_params=pltpu.CompilerParams(needs_layout_passes=False),
        scratch_types=[
            pltpu.VMEM((1, 128), _i32),
            pltpu.VMEM((1, 128), _i32),
            pltpu.VMEM((C4, H), _f32),
            pltpu.VMEM((C4, H), _f32),
            pltpu.VMEM_SHARED((NP, H), _f32),
            pltpu.SemaphoreType.DMA,
            pltpu.SemaphoreType.DMA,
            pltpu.SemaphoreType.DMA,
            pltpu.SemaphoreType.DMA,
        ],
    )(F, row2)


def _k4b_body(m0_hbm, m1_hbm, m2_hbm, row_hbm, pm_hbm,
              idx0, idx1, mv0, mv1, m0v0, m0v1, m1v0, m1v1, m2v0, m2v1,
              accM, sl0, sl1, ss0, ss1):
    cid = lax.axis_index("c")
    sid = lax.axis_index("s")
    wid = sid * NC + cid
    s0 = sid * SROWS
    idx = [idx0, idx1]
    mv = [mv0, mv1]
    m0v = [m0v0, m0v1]
    m1v = [m1v0, m1v1]
    m2v = [m2v0, m2v1]
    sl = [sl0, sl1]
    ss = [ss0, ss1]

    zero16 = jnp.zeros((16,), _f32)
    ones16 = jnp.ones((16,), _f32)
    lanes = lax.iota(_i32, 16)

    def zero_mv(i, carry):
        for jj in range(8):
            mv0[i, pl.ds(jj * 16, 16)] = zero16
            mv1[i, pl.ds(jj * 16, 16)] = zero16
        return carry

    lax.fori_loop(0, C4, zero_mv, 0)

    def zero_accm(k, carry):
        pltpu.sync_copy(mv0, accM.at[pl.ds(s0 + k * C4, C4)])
        return carry

    lax.fori_loop(0, SROWS // C4, zero_accm, 0)
    plsc.subcore_barrier()

    col0 = jnp.zeros((16,), _i32)
    col1 = col0 + 1
    col2 = col0 + 2
    col3 = col0 + 3

    def stage(k, bi):
        ch = wid * NCHUNK4 + k
        pltpu.async_copy(row_hbm.at[pl.ds(ch, 1)], idx[bi], sl[bi])
        pltpu.async_copy(m0_hbm.at[pl.ds(ch * C4, C4)], m0v[bi], sl[bi])
        pltpu.async_copy(m1_hbm.at[pl.ds(ch * C4, C4)], m1v[bi], sl[bi])
        pltpu.async_copy(m2_hbm.at[pl.ds(ch * C4, C4)], m2v[bi], sl[bi])

    def wait_load(bi):
        pltpu.make_async_copy(row_hbm.at[pl.ds(0, 1)], idx[bi], sl[bi]).wait()
        pltpu.make_async_copy(m0_hbm.at[pl.ds(0, C4)], m0v[bi], sl[bi]).wait()
        pltpu.make_async_copy(m1_hbm.at[pl.ds(0, C4)], m1v[bi], sl[bi]).wait()
        pltpu.make_async_copy(m2_hbm.at[pl.ds(0, C4)], m2v[bi], sl[bi]).wait()

    def pack(bi):
        def pack_body(gg, c2carry):
            base = gg * 16
            rows = base + lanes
            plsc.store_scatter(mv[bi], [rows, col0], m0v[bi][pl.ds(base, 16)])
            plsc.store_scatter(mv[bi], [rows, col1], m1v[bi][pl.ds(base, 16)])
            plsc.store_scatter(mv[bi], [rows, col2], m2v[bi][pl.ds(base, 16)])
            plsc.store_scatter(mv[bi], [rows, col3], ones16)
            return c2carry

        lax.fori_loop(0, C4 // 16, pack_body, 0)

    def scatter(bi):
        pltpu.async_copy(mv[bi], accM.at[idx[bi].at[0]], ss[bi], add=True)

    def drain_scatter(bi):
        pltpu.make_async_copy(mv[bi], accM.at[idx[bi].at[0]], ss[bi]).wait()

    stage(0, 0)
    wait_load(0)
    stage(1, 1)
    pack(0)
    scatter(0)

    def interior(tt, carry):
        k = 1 + 2 * tt
        wait_load(1)
        drain_scatter(0)
        stage(k + 1, 0)
        pack(1)
        scatter(1)
        wait_load(0)
        drain_scatter(1)
        stage(k + 2, 1)
        pack(0)
        scatter(0)
        return carry

    lax.fori_loop(0, (NCHUNK4 - 2) // 2, interior, 0)
    wait_load(1)
    pack(1)
    scatter(1)
    drain_scatter(0)
    drain_scatter(1)
    plsc.subcore_barrier()

    def fgroup(g, carry):
        r0 = s0 + g * C4
        pltpu.sync_copy(accM.at[pl.ds(r0, C4)], mv0)
        out0 = pl.multiple_of(cid * NP + r0, C4)
        pltpu.sync_copy(mv0, pm_hbm.at[pl.ds(out0, C4)])
        return carry

    lax.fori_loop(0, SROWS // C4, fgroup, 0)


def _k4b(M0, M1, M2, row2):
    mesh = plsc.VectorSubcoreMesh(core_axis_name="c", subcore_axis_name="s")
    return pl.kernel(
        _k4b_body,
        out_type=jax.ShapeDtypeStruct((2 * NP, H), _f32),
        mesh=mesh,
        compiler_params=pltpu.CompilerParams(needs_layout_passes=False),
        scratch_types=[
            pltpu.VMEM((1, 128), _i32),
            pltpu.VMEM((1, 128), _i32),
            pltpu.VMEM((C4, H), _f32),
            pltpu.VMEM((C4, H), _f32),
            pltpu.VMEM((C4,), _f32),
            pltpu.VMEM((C4,), _f32),
            pltpu.VMEM((C4,), _f32),
            pltpu.VMEM((C4,), _f32),
            pltpu.VMEM((C4,), _f32),
            pltpu.VMEM((C4,), _f32),
            pltpu.VMEM_SHARED((NP, H), _f32),
            pltpu.SemaphoreType.DMA,
            pltpu.SemaphoreType.DMA,
            pltpu.SemaphoreType.DMA,
            pltpu.SemaphoreType.DMA,
        ],
    )(M0, M1, M2, row2)


def _k5_body(he_ref, p0_ref, p1_ref, pm0_ref, pm1_ref,
             n1a_ref, n1b_ref, nb1_ref, n2_ref, nb2_ref,
             h_ref, v_ref):
    he = he_ref[...]
    agg = p0_ref[...] + p1_ref[...]
    t = jnp.dot(he.astype(_bf16), n1a_ref[...].astype(_bf16),
                preferred_element_type=_f32)
    t = t + jnp.dot(agg.astype(_bf16), n1b_ref[...].astype(_bf16),
                    preferred_element_type=_f32)
    t = jnp.maximum(t + nb1_ref[...], 0.0)
    h_ref[...] = he + jnp.dot(t.astype(_bf16), n2_ref[...].astype(_bf16),
                              preferred_element_type=_f32) + nb2_ref[...]

    vm = pm0_ref[...] + pm1_ref[...]
    cnt = jnp.maximum(vm[:, 3:4], 1.0)
    v1 = vm[:, 0:3] / cnt
    norm = jnp.sqrt(jnp.sum(v1 * v1, axis=1, keepdims=True))
    v3 = v1 / jnp.maximum(norm, 1e-12)
    v_ref[...] = jnp.concatenate([v3, jnp.zeros((NB, 13), _f32)], axis=1)


def _k5(he, P0, P1, PM0, PM1, n1a, n1b, nb1, n2, nb2):
    blk = pl.BlockSpec((NB, H), lambda i: (i, 0))
    sblk = pl.BlockSpec((NB, 16), lambda i: (i, 0))
    wblk = pl.BlockSpec((H, H), lambda i: (0, 0))
    vblk = pl.BlockSpec((1, H), lambda i: (0, 0))
    return pl.pallas_call(
        _k5_body,
        grid=(N // NB,),
        in_specs=[blk, blk, blk, blk, blk, wblk, wblk, vblk, wblk, vblk],
        out_specs=[blk, sblk],
        out_shape=[jax.ShapeDtypeStruct((N, H), _f32),
                   jax.ShapeDtypeStruct((N, 16), _f32)],
    )(he, P0, P1, PM0, PM1, n1a, n1b, nb1, n2, nb2)


@jax.jit
def _impl(h, x, edge_index, emb_W, emb_b, eW1, eb1, eW2, eb2,
          nW1, nb1, nW2, nb2, wW1, wb1, wW2, wb2):
    row = edge_index[0].astype(_i32)
    col = edge_index[1].astype(_i32)
    row2 = jnp.pad(row, (0, EP - E)).reshape(EP // 128, 128)
    col2 = jnp.pad(col, (0, EP - E)).reshape(EP // 128, 128)
    x0 = jnp.asarray(x[:, 0], _f32)
    x1 = jnp.asarray(x[:, 1], _f32)
    x2 = jnp.asarray(x[:, 2], _f32)

    he, A, B = _k1(h, emb_W, emb_b.reshape(1, H), eW1[:H], eW1[H:2 * H])
    G, D, C2, S2, PH = _k2(A, B, x0, x1, x2, row2, col2)
    nsh = (EP // EB, EB // 128, 128)
    F, M0, M1, M2 = _k3(
        G, D.reshape(nsh), C2.reshape(nsh), S2.reshape(nsh), PH.reshape(nsh),
        eW1[2 * H:2 * H + 1], eb1.reshape(1, H), eW2, eb2.reshape(1, H),
        wW1, wb1.reshape(1, H), wW2.reshape(1, H),
        jnp.broadcast_to(wb2.reshape(1, 1), (1, H)))
    P = _k4a(F, row2)
    P0, P1 = P[:NP], P[NP:]
    PM = _k4b(M0.reshape(EP), M1.reshape(EP), M2.reshape(EP), row2)
    PM0, PM1 = PM[:NP], PM[NP:]
    hout, vout = _k5(he, P0, P1, PM0, PM1,
                     nW1[:H], nW1[H:], nb1.reshape(1, H), nW2,
                     nb2.reshape(1, H))
    v = vout[:, :3].reshape(N, 1, 3)
    return (hout, x, v)


def kernel(h, x, edge_index, emb_W, emb_b, eW1, eb1, eW2, eb2,
           nW1, nb1, nW2, nb2, wW1, wb1, wW2, wb2):
    return _impl(h, x, edge_index, emb_W, emb_b, eW1, eb1, eW2, eb2,
                 nW1, nb1, nW2, nb2, wW1, wb1, wW2, wb2)


# speedup vs baseline: 3.1006x; 1.0046x over previous
import jax
import jax.numpy as jnp
import numpy as np
from jax import lax
from jax.experimental import pallas as pl
from jax.experimental.pallas import tpu as pltpu
from jax.experimental.pallas import tpu_sc as plsc

N = 10000
E = 320000
H = 128

NC = 2
NS = 16
NW = NC * NS
C = 256
IB = C // 128
EP = 327680
NCHUNK = EP // C // NW
NP = 10240
SROWS = NP // NS
EB = 512
NB = 1000

_f32 = jnp.float32
_i32 = jnp.int32
_bf16 = jnp.bfloat16


def _atan01(q):
    s = q * q
    return q * (0.99997726 + s * (-0.33262347 + s * (0.19354346 + s * (
        -0.11643287 + s * (0.05265332 + s * (-0.01172120))))))


def _k1_body(h_ref, embW_ref, embb_ref, w1a_ref, w1b_ref,
             he_ref, a_ref, b_ref):
    he = jnp.dot(h_ref[...].astype(_bf16), embW_ref[...].astype(_bf16),
                 preferred_element_type=_f32)
    he = he + embb_ref[...]
    he_ref[...] = he
    heb = he.astype(_bf16)
    a_ref[...] = jnp.dot(heb, w1a_ref[...].astype(_bf16),
                         preferred_element_type=_f32)
    b_ref[...] = jnp.dot(heb, w1b_ref[...].astype(_bf16),
                         preferred_element_type=_f32)


def _k1(h, emb_W, emb_b, w1a, w1b):
    blk = pl.BlockSpec((NB, H), lambda i: (i, 0))
    wblk = pl.BlockSpec((H, H), lambda i: (0, 0))
    vblk = pl.BlockSpec((1, H), lambda i: (0, 0))
    return pl.pallas_call(
        _k1_body,
        grid=(N // NB,),
        in_specs=[blk, wblk, vblk, wblk, wblk],
        out_specs=[blk, blk, blk],
        out_shape=[jax.ShapeDtypeStruct((N, H), _f32)] * 3,
    )(h, emb_W, emb_b, w1a, w1b)


C2 = 128
NCHUNK2 = EP // C2 // NW


def _k2_body(a_hbm, b_hbm, x0_hbm, x1_hbm, x2_hbm, rc_hbm,
             g_hbm, d_hbm, c2_hbm, s2_hbm, ph_hbm,
             idx0, idx1, av0, av1, bv0, bv1,
             x0_v, x1_v, x2_v, dv0, dv1, cv0, cv1, sv0, sv1, pv0, pv1,
             si0, si1, so0, so1):
    wid = lax.axis_index("s") * NC + lax.axis_index("c")
    idx = [idx0, idx1]
    av = [av0, av1]
    bv = [bv0, bv1]
    dv = [dv0, dv1]
    cv = [cv0, cv1]
    sv = [sv0, sv1]
    pv = [pv0, pv1]
    si = [si0, si1]
    so = [so0, so1]

    pltpu.sync_copy(x0_hbm, x0_v)
    pltpu.sync_copy(x1_hbm, x1_v)
    pltpu.sync_copy(x2_hbm, x2_v)
    lanes = lax.iota(_i32, 16)
    del lanes

    def stage_in(k, bi):
        ch = wid * NCHUNK2 + k
        pltpu.sync_copy(rc_hbm.at[pl.ds(2 * ch, 2)], idx[bi])
        pltpu.async_copy(a_hbm.at[idx[bi].at[0]], av[bi], si[bi])
        pltpu.async_copy(b_hbm.at[idx[bi].at[1]], bv[bi], si[bi])

    def drain_in(bi):
        pltpu.make_async_copy(a_hbm.at[idx[bi].at[0]], av[bi], si[bi]).wait()
        pltpu.make_async_copy(b_hbm.at[idx[bi].at[1]], bv[bi], si[bi]).wait()

    def compute(bi):
        def geo_body(gg, carry):
            off = gg * 16
            ir = idx[bi][0, pl.ds(off, 16)]
            ic = idx[bi][1, pl.ds(off, 16)]
            xd = plsc.load_gather(x0_v, [ir]) - plsc.load_gather(x0_v, [ic])
            yd = plsc.load_gather(x1_v, [ir]) - plsc.load_gather(x1_v, [ic])
            zd = plsc.load_gather(x2_v, [ir]) - plsc.load_gather(x2_v, [ic])
            x2 = xd * xd
            y2 = yd * yd
            rxy2 = x2 + y2
            d = rxy2 + zd * zd
            safe = jnp.maximum(rxy2, 1e-30)
            pos = rxy2 > 0.0
            c2 = jnp.where(pos, (x2 - y2) / safe, 1.0)
            s2 = jnp.where(pos, (2.0 * xd * yd) / safe, 0.0)
            ai = plsc.bitcast(safe, _i32)
            y = plsc.bitcast(_i32(0x5F3759DF) - (ai >> 1), _f32)
            y = y * (1.5 - 0.5 * safe * y * y)
            y = y * (1.5 - 0.5 * safe * y * y)
            y = y * (1.5 - 0.5 * safe * y * y)
            rxy = safe * y
            az = jnp.abs(zd)
            mx = jnp.maximum(az, rxy)
            mn = jnp.minimum(az, rxy)
            q = mn / jnp.maximum(mx, 1e-30)
            aa = _atan01(q)
            phi = jnp.sign(zd) * jnp.where(az > rxy,
                                           np.float32(np.pi / 2) - aa, aa)
            dv[bi][pl.ds(off, 16)] = d
            cv[bi][pl.ds(off, 16)] = c2
            sv[bi][pl.ds(off, 16)] = s2
            pv[bi][pl.ds(off, 16)] = phi
            return carry

        lax.fori_loop(0, C2 // 16, geo_body, 0)

        def add_body(i, carry):
            for jj in range(8):
                sl = pl.ds(jj * 16, 16)
                av[bi][i, sl] = av[bi][i, sl] + bv[bi][i, sl]
            return carry

        lax.fori_loop(0, C2, add_body, 0)

    def issue_outs(k, bi):
        e0 = (wid * NCHUNK2 + k) * C2
        pltpu.async_copy(av[bi], g_hbm.at[pl.ds(e0, C2)], so[bi])
        pltpu.async_copy(dv[bi], d_hbm.at[pl.ds(e0, C2)], so[bi])
        pltpu.async_copy(cv[bi], c2_hbm.at[pl.ds(e0, C2)], so[bi])
        pltpu.async_copy(sv[bi], s2_hbm.at[pl.ds(e0, C2)], so[bi])
        pltpu.async_copy(pv[bi], ph_hbm.at[pl.ds(e0, C2)], so[bi])

    def drain_out(bi):
        pltpu.make_async_copy(av[bi], g_hbm.at[pl.ds(0, C2)], so[bi]).wait()
        pltpu.make_async_copy(dv[bi], d_hbm.at[pl.ds(0, C2)], so[bi]).wait()
        pltpu.make_async_copy(cv[bi], c2_hbm.at[pl.ds(0, C2)], so[bi]).wait()
        pltpu.make_async_copy(sv[bi], s2_hbm.at[pl.ds(0, C2)], so[bi]).wait()
        pltpu.make_async_copy(pv[bi], ph_hbm.at[pl.ds(0, C2)], so[bi]).wait()

    stage_in(0, 0)
    drain_in(0)
    stage_in(1, 1)
    compute(0)
    issue_outs(0, 0)
    drain_in(1)
    drain_out(0)
    stage_in(2, 0)
    compute(1)
    issue_outs(1, 1)

    def interior(tt, carry):
        k0 = 2 * tt
        drain_in(0)
        drain_out(1)
        stage_in(k0 + 1, 1)
        compute(0)
        issue_outs(k0, 0)
        drain_in(1)
        drain_out(0)
        stage_in(k0 + 2, 0)
        compute(1)
        issue_outs(k0 + 1, 1)
        return carry

    lax.fori_loop(1, NCHUNK2 // 2 - 1, interior, 0)
    drain_in(0)
    drain_out(1)
    stage_in(NCHUNK2 - 1, 1)
    compute(0)
    issue_outs(NCHUNK2 - 2, 0)
    drain_in(1)
    compute(1)
    issue_outs(NCHUNK2 - 1, 1)
    drain_out(0)
    drain_out(1)


def _k2(A, B, x0, x1, x2, rc2):
    mesh = plsc.VectorSubcoreMesh(core_axis_name="c", subcore_axis_name="s")
    return pl.kernel(
        _k2_body,
        out_type=[jax.ShapeDtypeStruct((EP, H), _f32),
                  jax.ShapeDtypeStruct((EP,), _f32),
                  jax.ShapeDtypeStruct((EP,), _f32),
                  jax.ShapeDtypeStruct((EP,), _f32),
                  jax.ShapeDtypeStruct((EP,), _f32)],
        mesh=mesh,
        compiler_params=pltpu.CompilerParams(needs_layout_passes=False),
        scratch_types=[
            pltpu.VMEM((2, 128), _i32),
            pltpu.VMEM((2, 128), _i32),
            pltpu.VMEM((C2, H), _f32),
            pltpu.VMEM((C2, H), _f32),
            pltpu.VMEM((C2, H), _f32),
            pltpu.VMEM((C2, H), _f32),
            pltpu.VMEM((N,), _f32),
            pltpu.VMEM((N,), _f32),
            pltpu.VMEM((N,), _f32),
            pltpu.VMEM((C2,), _f32),
            pltpu.VMEM((C2,), _f32),
            pltpu.VMEM((C2,), _f32),
            pltpu.VMEM((C2,), _f32),
            pltpu.VMEM((C2,), _f32),
            pltpu.VMEM((C2,), _f32),
            pltpu.VMEM((C2,), _f32),
            pltpu.VMEM((C2,), _f32),
            pltpu.SemaphoreType.DMA,
            pltpu.SemaphoreType.DMA,
            pltpu.SemaphoreType.DMA,
            pltpu.SemaphoreType.DMA,
        ],
    )(A, B, x0, x1, x2, rc2)


def _k3_body(g_ref, d_ref, c2_ref, s2_ref, ph_ref, u_ref, b1_ref,
             w2_ref, b2_ref, ww1_ref, wb1_ref, wv2_ref, wb2_ref,
             f_ref, m0_ref, m1_ref, m2_ref):
    ident = (lax.broadcasted_iota(_i32, (128, 128), 0) ==
             lax.broadcasted_iota(_i32, (128, 128), 1)).astype(_f32)

    def unpack_col(pk):
        cols = [jnp.sum(jnp.broadcast_to(pk[i:i + 1, :], (128, 128)) * ident,
                        axis=1, keepdims=True) for i in range(EB // 128)]
        return jnp.concatenate(cols, axis=0)

    def pack_rows(colv):
        rows = [jnp.sum(colv[i * 128:(i + 1) * 128] * ident,
                        axis=0, keepdims=True) for i in range(EB // 128)]
        return jnp.concatenate(rows, axis=0)

    dpk = jnp.reshape(d_ref[...], (EB // 128, 128))
    cpk = jnp.reshape(c2_ref[...], (EB // 128, 128))
    spk = jnp.reshape(s2_ref[...], (EB // 128, 128))
    ppk = jnp.reshape(ph_ref[...], (EB // 128, 128))

    d = unpack_col(dpk)
    db = d.astype(_bf16).astype(_f32)
    ub = u_ref[...].astype(_bf16).astype(_f32)
    pre = g_ref[...] + db * ub + b1_ref[...]
    t1 = jnp.maximum(pre, 0.0)
    F0 = jnp.dot(t1.astype(_bf16), w2_ref[...].astype(_bf16),
                 preferred_element_type=_f32) + b2_ref[...]
    t2 = jnp.maximum(
        jnp.dot(F0.astype(_bf16), ww1_ref[...].astype(_bf16),
                preferred_element_type=_f32) + wb1_ref[...],
        0.0)
    t2b = t2.astype(_bf16).astype(_f32)
    wv2b = wv2_ref[...].astype(_bf16).astype(_f32)
    w = jnp.sum(t2b * wv2b, axis=1, keepdims=True) + wb2_ref[:, 0:1]

    eids = pl.program_id(0) * EB + lax.broadcasted_iota(_i32, (EB, 1), 0)
    f_ref[...] = jnp.where(eids < E, F0, 0.0)

    rr = lax.broadcasted_iota(_i32, (EB // 128, 128), 0)
    cc = lax.broadcasted_iota(_i32, (EB // 128, 128), 1)
    validpk = (pl.program_id(0) * EB + rr * 128 + cc) < E
    wpk = pack_rows(w)
    dwpk = jnp.where(validpk, dpk * wpk, 0.0)
    m0_ref[...] = jnp.reshape(dwpk * cpk, (1, EB // 128, 128))
    m1_ref[...] = jnp.reshape(dwpk * spk, (1, EB // 128, 128))
    m2_ref[...] = jnp.reshape(dwpk * ppk, (1, EB // 128, 128))


def _k3(G, D2, C22, S22, PH2, u, b1, eW2, b2, wW1, wb1, wv2, wb2):
    eblk = pl.BlockSpec((EB, H), lambda i: (i, 0))
    nblk = pl.BlockSpec((1, EB // 128, 128), lambda i: (i, 0, 0))
    wblk = pl.BlockSpec((H, H), lambda i: (0, 0))
    vblk = pl.BlockSpec((1, H), lambda i: (0, 0))
    nsh = jax.ShapeDtypeStruct((EP // EB, EB // 128, 128), _f32)
    return pl.pallas_call(
        _k3_body,
        grid=(EP // EB,),
        in_specs=[eblk, nblk, nblk, nblk, nblk,
                  vblk, vblk, wblk, vblk, wblk, vblk, vblk, vblk],
        out_specs=[eblk, nblk, nblk, nblk],
        out_shape=[jax.ShapeDtypeStruct((EP, H), _f32), nsh, nsh, nsh],
    )(G, D2, C22, S22, PH2, u, b1, eW2, b2, wW1, wb1, wv2, wb2)


C4 = 128
NCHUNK4 = EP // C4 // NW


def _k4a_body(f_hbm, row_hbm, p_hbm, idx0, idx1, fv0, fv1, accF,
              sl0, sl1, ss0, ss1):
    cid = lax.axis_index("c")
    sid = lax.axis_index("s")
    wid = sid * NC + cid
    s0 = sid * SROWS
    idx = [idx0, idx1]
    fv = [fv0, fv1]
    sl = [sl0, sl1]
    ss = [ss0, ss1]

    zero16 = jnp.zeros((16,), _f32)

    def zero_fv(i, carry):
        for jj in range(8):
            fv0[i, pl.ds(jj * 16, 16)] = zero16
        return carry

    lax.fori_loop(0, C4, zero_fv, 0)

    def zero_accf(k, carry):
        pltpu.sync_copy(fv0.at[pl.ds(0, SROWS // 8)],
                        accF.at[pl.ds(s0 + k * (SROWS // 8), SROWS // 8)])
        return carry

    lax.fori_loop(0, 8, zero_accf, 0)
    plsc.subcore_barrier()

    def stage(k, bi):
        ch = wid * NCHUNK4 + k
        pltpu.async_copy(row_hbm.at[pl.ds(ch, 1)], idx[bi], sl[bi])
        pltpu.async_copy(f_hbm.at[pl.ds(ch * C4, C4)], fv[bi], sl[bi])

    def wait_load(bi):
        pltpu.make_async_copy(row_hbm.at[pl.ds(0, 1)], idx[bi], sl[bi]).wait()
        pltpu.make_async_copy(f_hbm.at[pl.ds(0, C4)], fv[bi], sl[bi]).wait()

    def scatter(bi):
        pltpu.async_copy(fv[bi], accF.at[idx[bi].at[0]], ss[bi], add=True)

    def drain_scatter(bi):
        pltpu.make_async_copy(fv[bi], accF.at[idx[bi].at[0]], ss[bi]).wait()

    stage(0, 0)
    wait_load(0)
    stage(1, 1)
    scatter(0)

    def interior(tt, carry):
        k = 1 + 2 * tt
        wait_load(1)
        drain_scatter(0)
        stage(k + 1, 0)
        scatter(1)
        wait_load(0)
        drain_scatter(1)
        stage(k + 2, 1)
        scatter(0)
        return carry

    lax.fori_loop(0, (NCHUNK4 - 2) // 2, interior, 0)
    wait_load(1)
    scatter(1)
    drain_scatter(0)
    drain_scatter(1)
    plsc.subcore_barrier()

    def fgroup(g, carry):
        r0 = s0 + g * C4
        pltpu.sync_copy(accF.at[pl.ds(r0, C4)], fv0)
        out0 = pl.multiple_of(cid * NP + r0, C4)
        pltpu.sync_copy(fv0, p_hbm.at[pl.ds(out0, C4)])
        return carry

    lax.fori_loop(0, SROWS // C4, fgroup, 0)


def _k4a(F, row2):
    mesh = plsc.VectorSubcoreMesh(core_axis_name="c", subcore_axis_name="s")
    return pl.kernel(
        _k4a_body,
        out_type=jax.ShapeDtypeStruct((2 * NP, H), _f32),
        mesh=mesh,
        compiler_params=pltpu.CompilerParams(needs_layout_passes=False),
        scratch_types=[
            pltpu.VMEM((1, 128), _i32),
            pltpu.VMEM((1, 128), _i32),
            pltpu.VMEM((C4, H), _f32),
            pltpu.VMEM((C4, H), _f32),
            pltpu.VMEM_SHARED((NP, H), _f32),
            pltpu.SemaphoreType.DMA,
            pltpu.SemaphoreType.DMA,
            pltpu.SemaphoreType.DMA,
            pltpu.SemaphoreType.DMA,
        ],
    )(F, row2)


def _k4b_body(m0_hbm, m1_hbm, m2_hbm, row_hbm, pm_hbm,
              idx0, idx1, mv0, mv1, m0v0, m0v1, m1v0, m1v1, m2v0, m2v1,
              accM, sl0, sl1, ss0, ss1):
    cid = lax.axis_index("c")
    sid = lax.axis_index("s")
    wid = sid * NC + cid
    s0 = sid * SROWS
    idx = [idx0, idx1]
    mv = [mv0, mv1]
    m0v = [m0v0, m0v1]
    m1v = [m1v0, m1v1]
    m2v = [m2v0, m2v1]
    sl = [sl0, sl1]
    ss = [ss0, ss1]

    zero16 = jnp.zeros((16,), _f32)
    ones16 = jnp.ones((16,), _f32)
    lanes = lax.iota(_i32, 16)

    def zero_mv(i, carry):
        for jj in range(8):
            mv0[i, pl.ds(jj * 16, 16)] = zero16
            mv1[i, pl.ds(jj * 16, 16)] = zero16
        return carry

    lax.fori_loop(0, C4, zero_mv, 0)

    def zero_accm(k, carry):
        pltpu.sync_copy(mv0, accM.at[pl.ds(s0 + k * C4, C4)])
        return carry

    lax.fori_loop(0, SROWS // C4, zero_accm, 0)
    plsc.subcore_barrier()

    col0 = jnp.zeros((16,), _i32)
    col1 = col0 + 1
    col2 = col0 + 2
    col3 = col0 + 3

    def stage(k, bi):
        ch = wid * NCHUNK4 + k
        pltpu.async_copy(row_hbm.at[pl.ds(ch, 1)], idx[bi], sl[bi])
        pltpu.async_copy(m0_hbm.at[pl.ds(ch * C4, C4)], m0v[bi], sl[bi])
        pltpu.async_copy(m1_hbm.at[pl.ds(ch * C4, C4)], m1v[bi], sl[bi])
        pltpu.async_copy(m2_hbm.at[pl.ds(ch * C4, C4)], m2v[bi], sl[bi])

    def wait_load(bi):
        pltpu.make_async_copy(row_hbm.at[pl.ds(0, 1)], idx[bi], sl[bi]).wait()
        pltpu.make_async_copy(m0_hbm.at[pl.ds(0, C4)], m0v[bi], sl[bi]).wait()
        pltpu.make_async_copy(m1_hbm.at[pl.ds(0, C4)], m1v[bi], sl[bi]).wait()
        pltpu.make_async_copy(m2_hbm.at[pl.ds(0, C4)], m2v[bi], sl[bi]).wait()

    def pack(bi):
        def pack_body(gg, c2carry):
            base = gg * 16
            rows = base + lanes
            plsc.store_scatter(mv[bi], [rows, col0], m0v[bi][pl.ds(base, 16)])
            plsc.store_scatter(mv[bi], [rows, col1], m1v[bi][pl.ds(base, 16)])
            plsc.store_scatter(mv[bi], [rows, col2], m2v[bi][pl.ds(base, 16)])
            plsc.store_scatter(mv[bi], [rows, col3], ones16)
            return c2carry

        lax.fori_loop(0, C4 // 16, pack_body, 0)

    def scatter(bi):
        pltpu.async_copy(mv[bi], accM.at[idx[bi].at[0]], ss[bi], add=True)

    def drain_scatter(bi):
        pltpu.make_async_copy(mv[bi], accM.at[idx[bi].at[0]], ss[bi]).wait()

    stage(0, 0)
    wait_load(0)
    stage(1, 1)
    pack(0)
    scatter(0)

    def interior(tt, carry):
        k = 1 + 2 * tt
        wait_load(1)
        drain_scatter(0)
        stage(k + 1, 0)
        pack(1)
        scatter(1)
        wait_load(0)
        drain_scatter(1)
        stage(k + 2, 1)
        pack(0)
        scatter(0)
        return carry

    lax.fori_loop(0, (NCHUNK4 - 2) // 2, interior, 0)
    wait_load(1)
    pack(1)
    scatter(1)
    drain_scatter(0)
    drain_scatter(1)
    plsc.subcore_barrier()

    def fgroup(g, carry):
        r0 = s0 + g * C4
        pltpu.sync_copy(accM.at[pl.ds(r0, C4)], mv0)
        out0 = pl.multiple_of(cid * NP + r0, C4)
        pltpu.sync_copy(mv0, pm_hbm.at[pl.ds(out0, C4)])
        return carry

    lax.fori_loop(0, SROWS // C4, fgroup, 0)


def _k4b(M0, M1, M2, row2):
    mesh = plsc.VectorSubcoreMesh(core_axis_name="c", subcore_axis_name="s")
    return pl.kernel(
        _k4b_body,
        out_type=jax.ShapeDtypeStruct((2 * NP, H), _f32),
        mesh=mesh,
        compiler_params=pltpu.CompilerParams(needs_layout_passes=False),
        scratch_types=[
            pltpu.VMEM((1, 128), _i32),
            pltpu.VMEM((1, 128), _i32),
            pltpu.VMEM((C4, H), _f32),
            pltpu.VMEM((C4, H), _f32),
            pltpu.VMEM((C4,), _f32),
            pltpu.VMEM((C4,), _f32),
            pltpu.VMEM((C4,), _f32),
            pltpu.VMEM((C4,), _f32),
            pltpu.VMEM((C4,), _f32),
            pltpu.VMEM((C4,), _f32),
            pltpu.VMEM_SHARED((NP, H), _f32),
            pltpu.SemaphoreType.DMA,
            pltpu.SemaphoreType.DMA,
            pltpu.SemaphoreType.DMA,
            pltpu.SemaphoreType.DMA,
        ],
    )(M0, M1, M2, row2)


def _k5_body(he_ref, p0_ref, p1_ref, pm0_ref, pm1_ref,
             n1a_ref, n1b_ref, nb1_ref, n2_ref, nb2_ref,
             h_ref, v_ref):
    he = he_ref[...]
    agg = p0_ref[...] + p1_ref[...]
    t = jnp.dot(he.astype(_bf16), n1a_ref[...].astype(_bf16),
                preferred_element_type=_f32)
    t = t + jnp.dot(agg.astype(_bf16), n1b_ref[...].astype(_bf16),
                    preferred_element_type=_f32)
    t = jnp.maximum(t + nb1_ref[...], 0.0)
    h_ref[...] = he + jnp.dot(t.astype(_bf16), n2_ref[...].astype(_bf16),
                              preferred_element_type=_f32) + nb2_ref[...]

    vm = pm0_ref[...] + pm1_ref[...]
    cnt = jnp.maximum(vm[:, 3:4], 1.0)
    v1 = vm[:, 0:3] / cnt
    norm = jnp.sqrt(jnp.sum(v1 * v1, axis=1, keepdims=True))
    v3 = v1 / jnp.maximum(norm, 1e-12)
    v_ref[...] = jnp.concatenate([v3, jnp.zeros((NB, 13), _f32)], axis=1)


def _k5(he, P0, P1, PM0, PM1, n1a, n1b, nb1, n2, nb2):
    blk = pl.BlockSpec((NB, H), lambda i: (i, 0))
    sblk = pl.BlockSpec((NB, 16), lambda i: (i, 0))
    wblk = pl.BlockSpec((H, H), lambda i: (0, 0))
    vblk = pl.BlockSpec((1, H), lambda i: (0, 0))
    return pl.pallas_call(
        _k5_body,
        grid=(N // NB,),
        in_specs=[blk, blk, blk, blk, blk, wblk, wblk, vblk, wblk, vblk],
        out_specs=[blk, sblk],
        out_shape=[jax.ShapeDtypeStruct((N, H), _f32),
                   jax.ShapeDtypeStruct((N, 16), _f32)],
    )(he, P0, P1, PM0, PM1, n1a, n1b, nb1, n2, nb2)


@jax.jit
def _impl(h, x, edge_index, emb_W, emb_b, eW1, eb1, eW2, eb2,
          nW1, nb1, nW2, nb2, wW1, wb1, wW2, wb2):
    row = edge_index[0].astype(_i32)
    col = edge_index[1].astype(_i32)
    row2 = jnp.pad(row, (0, EP - E)).reshape(EP // 128, 128)
    col2 = jnp.pad(col, (0, EP - E)).reshape(EP // 128, 128)
    x0 = jnp.asarray(x[:, 0], _f32)
    x1 = jnp.asarray(x[:, 1], _f32)
    x2 = jnp.asarray(x[:, 2], _f32)

    he, A, B = _k1(h, emb_W, emb_b.reshape(1, H), eW1[:H], eW1[H:2 * H])
    rc2 = jnp.stack([row2, col2], axis=1).reshape(2 * (EP // 128), 128)
    G, D, C2v, S2v, PH = _k2(A, B, x0, x1, x2, rc2)
    nsh = (EP // EB, EB // 128, 128)
    F, M0, M1, M2 = _k3(
        G, D.reshape(nsh), C2v.reshape(nsh), S2v.reshape(nsh), PH.reshape(nsh),
        eW1[2 * H:2 * H + 1], eb1.reshape(1, H), eW2, eb2.reshape(1, H),
        wW1, wb1.reshape(1, H), wW2.reshape(1, H),
        jnp.broadcast_to(wb2.reshape(1, 1), (1, H)))
    P = _k4a(F, row2)
    P0, P1 = P[:NP], P[NP:]
    PM = _k4b(M0.reshape(EP), M1.reshape(EP), M2.reshape(EP), row2)
    PM0, PM1 = PM[:NP], PM[NP:]
    hout, vout = _k5(he, P0, P1, PM0, PM1,
                     nW1[:H], nW1[H:], nb1.reshape(1, H), nW2,
                     nb2.reshape(1, H))
    v = vout[:, :3].reshape(N, 1, 3)
    return (hout, x, v)


def kernel(h, x, edge_index, emb_W, emb_b, eW1, eb1, eW2, eb2,
           nW1, nb1, nW2, nb2, wW1, wb1, wW2, wb2):
    return _impl(h, x, edge_index, emb_W, emb_b, eW1, eb1, eW2, eb2,
                 nW1, nb1, nW2, nb2, wW1, wb1, wW2, wb2)
